# Initial kernel scaffold; baseline (speedup 1.0000x reference)
import jax
import jax.numpy as jnp
from jax.experimental import pallas as pl


def kernel(scores, bbox_deltas, im_info):
    raise NotImplementedError("write your pallas kernel here")


# speedup vs baseline: 12.8745x; 12.8745x over previous
import numpy as np
import jax
import jax.numpy as jnp
from jax import lax
from jax.experimental import pallas as pl
from jax.experimental.pallas import tpu as pltpu

FEAT_STRIDE = 16
NUM_ANCHORS = 9
PRE_NMS_TOPN = 6000
POST_NMS_TOPN = 300
NMS_THRESH = 0.7
MIN_SIZE = 16.0
H, W = 50, 80
N = H * W * NUM_ANCHORS
ROWS = 288
NPAD = ROWS * 128
NEG = -1e9


def _whctrs_k(anchor):
    w = anchor[2] - anchor[0] + 1.0
    h = anchor[3] - anchor[1] + 1.0
    x_ctr = anchor[0] + 0.5 * (w - 1)
    y_ctr = anchor[1] + 0.5 * (h - 1)
    return w, h, x_ctr, y_ctr


def _mkanchors_k(ws, hs, x_ctr, y_ctr):
    ws = ws[:, np.newaxis]
    hs = hs[:, np.newaxis]
    return np.hstack((x_ctr - 0.5 * (ws - 1), y_ctr - 0.5 * (hs - 1),
                      x_ctr + 0.5 * (ws - 1), y_ctr + 0.5 * (hs - 1)))


def _gen_anchor_table(base_size=16, ratios=np.array([0.5, 1.0, 2.0]),
                      scales=np.array([8.0, 16.0, 32.0])):
    base_anchor = np.array([1.0, 1.0, base_size, base_size]) - 1.0
    w, h, x_ctr, y_ctr = _whctrs_k(base_anchor)
    size_ratios = (w * h) / ratios
    ws0 = np.round(np.sqrt(size_ratios))
    hs0 = np.round(ws0 * ratios)
    ratio_anchors = _mkanchors_k(ws0, hs0, x_ctr, y_ctr)
    outs = []
    for i in range(ratio_anchors.shape[0]):
        w, h, x_ctr, y_ctr = _whctrs_k(ratio_anchors[i, :])
        outs.append(_mkanchors_k(w * scales, h * scales, x_ctr, y_ctr))
    return np.vstack(outs).astype(np.float32)


def _anchor_consts():
    base = _gen_anchor_table().astype(np.float64)
    sx = (np.arange(W) * FEAT_STRIDE).astype(np.float64)
    sy = (np.arange(H) * FEAT_STRIDE).astype(np.float64)
    SX, SY = np.meshgrid(sx, sy)
    shifts = np.stack([SX.ravel(), SY.ravel(), SX.ravel(), SY.ravel()], axis=1)
    anchors = (base[None, :, :] + shifts[:, None, :]).reshape(-1, 4)
    widths = anchors[:, 2] - anchors[:, 0] + 1.0
    heights = anchors[:, 3] - anchors[:, 1] + 1.0
    ctr_x = anchors[:, 0] + 0.5 * widths
    ctr_y = anchors[:, 1] + 0.5 * heights

    def padr(v):
        out = np.zeros((NPAD,), np.float32)
        out[:N] = v.astype(np.float32)
        return out.reshape(ROWS, 128)

    return padr(widths), padr(heights), padr(ctr_x), padr(ctr_y)


_WA, _HA, _CXA, _CYA = _anchor_consts()


def _body(sc_ref, dx_ref, dy_ref, dw_ref, dh_ref, wa_ref, ha_ref, cxa_ref,
          cya_ref, im_ref, out_ref, x1_ref, y1_ref, x2_ref, y2_ref, ar_ref,
          s_ref):
    im_h = im_ref[0, 0]
    im_w = im_ref[0, 1]
    im_scale = im_ref[0, 2]

    wa = wa_ref[...]
    ha = ha_ref[...]
    dx = dx_ref[...]
    dy = dy_ref[...]
    dw = jnp.clip(dw_ref[...], -10.0, 10.0)
    dh = jnp.clip(dh_ref[...], -10.0, 10.0)
    pcx = dx * wa + cxa_ref[...]
    pcy = dy * ha + cya_ref[...]
    pw = jnp.exp(dw) * wa
    ph = jnp.exp(dh) * ha
    x1 = jnp.clip(pcx - 0.5 * pw, 0.0, im_w - 1.0)
    y1 = jnp.clip(pcy - 0.5 * ph, 0.0, im_h - 1.0)
    x2 = jnp.clip(pcx + 0.5 * pw, 0.0, im_w - 1.0)
    y2 = jnp.clip(pcy + 0.5 * ph, 0.0, im_h - 1.0)
    ws = x2 - x1 + 1.0
    hs = y2 - y1 + 1.0
    min_sz = MIN_SIZE * im_scale
    keep = (ws >= min_sz) & (hs >= min_sz)

    ri = lax.broadcasted_iota(jnp.int32, (ROWS, 128), 0)
    ci = lax.broadcasted_iota(jnp.int32, (ROWS, 128), 1)
    n_i = ri * 128 + ci
    valid = n_i < N

    s0 = jnp.where(keep, sc_ref[...], jnp.float32(NEG))
    s0 = jnp.where(valid, s0, -jnp.inf)

    ks = lax.bitcast_convert_type(s0, jnp.int32)
    key_i = ks ^ ((ks >> 31) & jnp.int32(0x7FFFFFFF))
    ku = lax.bitcast_convert_type(key_i, jnp.uint32) ^ jnp.uint32(0x80000000)

    K = jnp.float32(PRE_NMS_TOPN)

    def tau_step(t, prefix):
        b = (31 - t).astype(jnp.uint32)
        cand = prefix | (jnp.uint32(1) << b)
        cnt = jnp.sum(jnp.where(ku >= cand, 1.0, 0.0))
        return jnp.where(cnt >= K, cand, prefix)

    tau = lax.fori_loop(0, 32, tau_step, jnp.uint32(0))

    c_gt = jnp.sum(jnp.where(ku > tau, 1.0, 0.0))
    needed = K - c_gt
    eqm = ku == tau

    def cut_step(t, prefix):
        cand = prefix | (jnp.int32(1) << (16 - t))
        cnt = jnp.sum(jnp.where(eqm & (n_i < cand), 1.0, 0.0))
        return jnp.where(cnt < needed, cand, prefix)

    tcut = lax.fori_loop(0, 17, cut_step, jnp.int32(0))
    elig = (ku > tau) | (eqm & (n_i <= tcut) & (needed >= 1.0))

    x1_ref[...] = x1
    y1_ref[...] = y1
    x2_ref[...] = x2
    y2_ref[...] = y2
    ar_ref[...] = ws * hs
    s_ref[...] = jnp.where(elig, s0, -jnp.inf)

    nf = ri.astype(jnp.float32) * 128.0 + ci.astype(jnp.float32)
    li = lax.broadcasted_iota(jnp.int32, (1, 128), 1)

    def nms_step(step, i0):
        s = s_ref[...]
        m = jnp.max(s)
        idx = jnp.min(jnp.where(s == m, nf, jnp.float32(NPAD))).astype(jnp.int32)
        i0n = jnp.where(step == 0, idx, i0)
        sel = jnp.where(m == jnp.float32(NEG), i0n, idx)
        r = sel // 128
        c = sel % 128
        lm = li == c
        bx1 = jnp.sum(jnp.where(lm, x1_ref[pl.ds(r, 1), :], 0.0))
        by1 = jnp.sum(jnp.where(lm, y1_ref[pl.ds(r, 1), :], 0.0))
        bx2 = jnp.sum(jnp.where(lm, x2_ref[pl.ds(r, 1), :], 0.0))
        by2 = jnp.sum(jnp.where(lm, y2_ref[pl.ds(r, 1), :], 0.0))
        bar = jnp.sum(jnp.where(lm, ar_ref[pl.ds(r, 1), :], 0.0))
        ax1 = x1_ref[...]
        ay1 = y1_ref[...]
        ax2 = x2_ref[...]
        ay2 = y2_ref[...]
        ar = ar_ref[...]
        w = jnp.maximum(0.0, jnp.minimum(bx2, ax2) - jnp.maximum(bx1, ax1) + 1.0)
        h = jnp.maximum(0.0, jnp.minimum(by2, ay2) - jnp.maximum(by1, ay1) + 1.0)
        inter = w * h
        iou = inter / (bar + ar - inter)
        s_ref[...] = jnp.where(iou > jnp.float32(NMS_THRESH),
                               jnp.minimum(s, jnp.float32(NEG)), s)
        rv = jnp.zeros((1, 128), jnp.float32)
        rv = jnp.where(li == 1, bx1, rv)
        rv = jnp.where(li == 2, by1, rv)
        rv = jnp.where(li == 3, bx2, rv)
        rv = jnp.where(li == 4, by2, rv)
        out_ref[pl.ds(step, 1), :] = rv
        return i0n

    lax.fori_loop(0, POST_NMS_TOPN, nms_step, jnp.int32(0))


def kernel(scores, bbox_deltas, im_info):
    sfg = jnp.transpose(scores[0, NUM_ANCHORS:], (1, 2, 0)).reshape(-1)
    dl = jnp.transpose(bbox_deltas[0], (1, 2, 0)).reshape(-1, 4)

    def pad2(v):
        return jnp.concatenate(
            [v, jnp.zeros((NPAD - N,), jnp.float32)]).reshape(ROWS, 128)

    out = pl.pallas_call(
        _body,
        out_shape=jax.ShapeDtypeStruct((POST_NMS_TOPN, 128), jnp.float32),
        in_specs=[pl.BlockSpec(memory_space=pltpu.MemorySpace.VMEM)] * 9
        + [pl.BlockSpec(memory_space=pltpu.MemorySpace.SMEM)],
        out_specs=pl.BlockSpec(memory_space=pltpu.MemorySpace.VMEM),
        scratch_shapes=[pltpu.VMEM((ROWS, 128), jnp.float32)] * 6,
    )(pad2(sfg), pad2(dl[:, 0]), pad2(dl[:, 1]), pad2(dl[:, 2]),
      pad2(dl[:, 3]), jnp.asarray(_WA), jnp.asarray(_HA), jnp.asarray(_CXA),
      jnp.asarray(_CYA), im_info)
    return out[:, :5]
